# baseline (speedup 1.0000x reference)
import functools

import jax
import jax.numpy as jnp
from jax import lax
from jax.experimental import pallas as pl
from jax.experimental.pallas import tpu as pltpu
from jax.experimental.pallas import tpu_sc as plsc

NTA = 16384
O = 2048
NMAX = 2048
E = 8

B = 256
PAD_N = NTA + E * B
NBLK = PAD_N // B
G = 4
SUB = NBLK // G
ROWS_G = SUB * B
CHUNK = 24

_NC, _NS = 2, 16
_NW = _NC * _NS


def _sc_move_body(gather, lin_hbm, idx_hbm, rnd_hbm, idx_v, rows0, rows1,
                  sem0, sem1):
    rows_per_w = ROWS_G // _NW // CHUNK
    wid = lax.axis_index("s") * _NC + lax.axis_index("c")
    base = wid * (rows_per_w * CHUNK)
    pltpu.sync_copy(idx_hbm.at[wid], idx_v)

    bufs = (rows0, rows1)
    sems = (sem0, sem1)
    second = [None, None]
    for j in range(rows_per_w):
        bi = j % 2
        if second[bi] is not None:
            second[bi].wait()
        lin = lin_hbm.at[pl.ds(base + j * CHUNK, CHUNK)]
        rnd = rnd_hbm.at[idx_v.at[j]]
        src, dst = (rnd, lin) if gather else (lin, rnd)
        pltpu.async_copy(src, bufs[bi], sems[bi]).wait()
        second[bi] = pltpu.async_copy(bufs[bi], dst, sems[bi])
    second[0].wait()
    second[1].wait()


def _mesh():
    return plsc.VectorSubcoreMesh(core_axis_name="c", subcore_axis_name="s")


def _sc_scratch(width, dtype):
    return [
        pltpu.VMEM((ROWS_G // _NW // CHUNK, CHUNK), jnp.int32),
        pltpu.VMEM((CHUNK, width), dtype),
        pltpu.VMEM((CHUNK, width), dtype),
        pltpu.SemaphoreType.DMA,
        pltpu.SemaphoreType.DMA,
    ]


def _sc_gather(table, idx3d):
    def body(table_hbm, idx_hbm, out_hbm, *scratch):
        _sc_move_body(True, out_hbm, idx_hbm, table_hbm, *scratch)

    return pl.kernel(
        body,
        out_type=jax.ShapeDtypeStruct((ROWS_G, O), jnp.float32),
        mesh=_mesh(),
        scratch_types=_sc_scratch(O, jnp.float32),
    )(table, idx3d)


def _sc_scatter(y, idx3d, out_ref):
    def body(y_hbm, idx_hbm, o_ref, *scratch):
        _sc_move_body(False, y_hbm, idx_hbm, o_ref, *scratch)

    pl.kernel(
        body,
        out_type=(),
        mesh=_mesh(),
        scratch_types=_sc_scratch(O // 2, jnp.int32),
    )(y, idx3d, out_ref)


def _alloc_body(o_ref):
    pass


def _alloc_out():
    return pl.pallas_call(
        _alloc_body,
        out_shape=jax.ShapeDtypeStruct((NTA, NMAX // 2), jnp.int32),
        out_specs=pl.BlockSpec(memory_space=pltpu.MemorySpace.HBM),
    )()


def _mm_body(expert_ref, x_ref, w_ref, b_ref, o_ref):
    o_ref[...] = (jnp.dot(x_ref[...], w_ref[0]) + b_ref[0]
                  ).astype(jnp.bfloat16)


def _expert_matmul(rho_g, W, b3, block_expert_g):
    grid_spec = pltpu.PrefetchScalarGridSpec(
        num_scalar_prefetch=1,
        grid=(SUB,),
        in_specs=[
            pl.BlockSpec((B, O), lambda i, e_ref: (i, 0)),
            pl.BlockSpec((1, O, NMAX), lambda i, e_ref: (e_ref[i], 0, 0)),
            pl.BlockSpec((1, 1, NMAX), lambda i, e_ref: (e_ref[i], 0, 0)),
        ],
        out_specs=pl.BlockSpec((B, NMAX), lambda i, e_ref: (i, 0)),
    )
    return pl.pallas_call(
        _mm_body,
        grid_spec=grid_spec,
        out_shape=jax.ShapeDtypeStruct((ROWS_G, NMAX), jnp.bfloat16),
        compiler_params=pltpu.CompilerParams(
            dimension_semantics=("arbitrary",)),
    )(block_expert_g, rho_g, W, b3)


def kernel(rho, symbols, W, b):
    sym = symbols.astype(jnp.int32)

    i_arr = jnp.arange(NTA, dtype=jnp.int32)
    scramble = (i_arr * 40503) & (NTA - 1)
    sidx = jnp.argsort(sym * NTA + scramble).astype(jnp.int32)
    e_ids = jnp.arange(E, dtype=jnp.int32)
    counts = (sym[:, None] == e_ids[None, :]).sum(0).astype(jnp.int32)
    starts = jnp.cumsum(counts) - counts
    padded_counts = ((counts + B - 1) // B) * B
    pcum = jnp.cumsum(padded_counts)
    pstarts = pcum - padded_counts
    total = pcum[-1]

    q_arr = jnp.arange(PAD_N, dtype=jnp.int32)
    qq = jnp.where(q_arr < total, q_arr, q_arr - total)
    e_q = jnp.minimum((qq[:, None] >= pcum[None, :]).sum(1), E - 1)
    r_q = qq - pstarts[e_q]
    src_p = starts[e_q] + r_q % jnp.maximum(counts[e_q], 1)
    gidx = sidx[src_p]
    gidx4d = gidx.reshape(G, _NW, ROWS_G // _NW // CHUNK, CHUNK)

    jb = jnp.arange(NBLK, dtype=jnp.int32) * B
    jb = jnp.where(jb < total, jb, jb - total)
    block_expert = jnp.minimum((jb[:, None] >= pcum[None, :]).sum(1), E - 1
                               ).astype(jnp.int32)

    b3 = b.reshape(E, 1, NMAX)
    out_ref = jax.new_ref(_alloc_out())
    for g in range(G):
        idx_g = gidx4d[g]
        rho_g = _sc_gather(rho, idx_g)
        y_g = _expert_matmul(rho_g, W, b3, block_expert[g * SUB:(g + 1) * SUB])
        y32 = lax.bitcast_convert_type(
            y_g.reshape(ROWS_G, NMAX // 2, 2), jnp.int32)
        _sc_scatter(y32, idx_g, out_ref)
    out16 = lax.bitcast_convert_type(out_ref[...], jnp.bfloat16)
    return out16.reshape(NTA, NMAX).astype(jnp.float32)


# speedup vs baseline: 3.7185x; 3.7185x over previous
import functools

import jax
import jax.numpy as jnp
from jax import lax
from jax.experimental import pallas as pl
from jax.experimental.pallas import tpu as pltpu
from jax.experimental.pallas import tpu_sc as plsc

NTA = 16384
O = 2048
NMAX = 2048
E = 8

B = 256
PAD_N = NTA + E * B
NBLK = PAD_N // B
G = 4
SUB = NBLK // G
ROWS_G = SUB * B
CHUNK = 24

_NC, _NS = 2, 16
_NW = _NC * _NS


def _sc_move_body(gather, lin_hbm, idx_hbm, rnd_hbm, idx_v, rows0, rows1,
                  sem0, sem1):
    rows_per_w = ROWS_G // _NW // CHUNK
    wid = lax.axis_index("s") * _NC + lax.axis_index("c")
    base = wid * (rows_per_w * CHUNK)
    pltpu.sync_copy(idx_hbm.at[wid], idx_v)

    bufs = (rows0, rows1)
    sems = (sem0, sem1)
    second = [None, None]
    for j in range(rows_per_w):
        bi = j % 2
        if second[bi] is not None:
            second[bi].wait()
        lin = lin_hbm.at[pl.ds(base + j * CHUNK, CHUNK)]
        rnd = rnd_hbm.at[idx_v.at[j]]
        src, dst = (rnd, lin) if gather else (lin, rnd)
        pltpu.async_copy(src, bufs[bi], sems[bi]).wait()
        second[bi] = pltpu.async_copy(bufs[bi], dst, sems[bi])
    second[0].wait()
    second[1].wait()


def _mesh():
    return plsc.VectorSubcoreMesh(core_axis_name="c", subcore_axis_name="s")


def _sc_scratch(width, dtype):
    return [
        pltpu.VMEM((ROWS_G // _NW // CHUNK, CHUNK), jnp.int32),
        pltpu.VMEM((CHUNK, width), dtype),
        pltpu.VMEM((CHUNK, width), dtype),
        pltpu.SemaphoreType.DMA,
        pltpu.SemaphoreType.DMA,
    ]


def _sc_gather(table, idx3d):
    def body(table_hbm, idx_hbm, out_hbm, *scratch):
        _sc_move_body(True, out_hbm, idx_hbm, table_hbm, *scratch)

    return pl.kernel(
        body,
        out_type=jax.ShapeDtypeStruct((ROWS_G, O), jnp.float32),
        mesh=_mesh(),
        scratch_types=_sc_scratch(O, jnp.float32),
    )(table, idx3d)


def _sc_scatter(y, idx3d, out_ref):
    def body(y_hbm, idx_hbm, o_ref, *scratch):
        _sc_move_body(False, y_hbm, idx_hbm, o_ref, *scratch)

    pl.kernel(
        body,
        out_type=(),
        mesh=_mesh(),
        scratch_types=_sc_scratch(O, jnp.float32),
    )(y, idx3d, out_ref)


def _alloc_body(o_ref):
    pass


def _alloc_out():
    return pl.pallas_call(
        _alloc_body,
        out_shape=jax.ShapeDtypeStruct((NTA, NMAX), jnp.float32),
        out_specs=pl.BlockSpec(memory_space=pltpu.MemorySpace.HBM),
    )()


def _mm_body(expert_ref, x_ref, w_ref, b_ref, o_ref):
    o_ref[...] = jnp.dot(x_ref[...], w_ref[0]) + b_ref[0]


def _expert_matmul(rho_g, W, b3, block_expert_g):
    grid_spec = pltpu.PrefetchScalarGridSpec(
        num_scalar_prefetch=1,
        grid=(SUB,),
        in_specs=[
            pl.BlockSpec((B, O), lambda i, e_ref: (i, 0)),
            pl.BlockSpec((1, O, NMAX), lambda i, e_ref: (e_ref[i], 0, 0)),
            pl.BlockSpec((1, 1, NMAX), lambda i, e_ref: (e_ref[i], 0, 0)),
        ],
        out_specs=pl.BlockSpec((B, NMAX), lambda i, e_ref: (i, 0)),
    )
    return pl.pallas_call(
        _mm_body,
        grid_spec=grid_spec,
        out_shape=jax.ShapeDtypeStruct((ROWS_G, NMAX), jnp.float32),
        compiler_params=pltpu.CompilerParams(
            dimension_semantics=("arbitrary",)),
    )(block_expert_g, rho_g, W, b3)


def kernel(rho, symbols, W, b):
    sym = symbols.astype(jnp.int32)

    i_arr = jnp.arange(NTA, dtype=jnp.int32)
    scramble = (i_arr * 40503) & (NTA - 1)
    sidx = jnp.argsort(sym * NTA + scramble).astype(jnp.int32)
    e_ids = jnp.arange(E, dtype=jnp.int32)
    counts = (sym[:, None] == e_ids[None, :]).sum(0).astype(jnp.int32)
    starts = jnp.cumsum(counts) - counts
    padded_counts = ((counts + B - 1) // B) * B
    pcum = jnp.cumsum(padded_counts)
    pstarts = pcum - padded_counts
    total = pcum[-1]

    q_arr = jnp.arange(PAD_N, dtype=jnp.int32)
    qq = jnp.where(q_arr < total, q_arr, q_arr - total)
    e_q = jnp.minimum((qq[:, None] >= pcum[None, :]).sum(1), E - 1)
    r_q = qq - pstarts[e_q]
    src_p = starts[e_q] + r_q % jnp.maximum(counts[e_q], 1)
    gidx = sidx[src_p]
    gidx4d = gidx.reshape(G, _NW, ROWS_G // _NW // CHUNK, CHUNK)

    jb = jnp.arange(NBLK, dtype=jnp.int32) * B
    jb = jnp.where(jb < total, jb, jb - total)
    block_expert = jnp.minimum((jb[:, None] >= pcum[None, :]).sum(1), E - 1
                               ).astype(jnp.int32)

    b3 = b.reshape(E, 1, NMAX)
    out_ref = jax.new_ref(_alloc_out())
    for g in range(G):
        idx_g = gidx4d[g]
        rho_g = _sc_gather(rho, idx_g)
        y_g = _expert_matmul(rho_g, W, b3, block_expert[g * SUB:(g + 1) * SUB])
        _sc_scatter(y_g, idx_g, out_ref)
    return out_ref[...]
